# Initial kernel scaffold; baseline (speedup 1.0000x reference)
import functools

import jax
import jax.numpy as jnp
from jax import lax
from jax.experimental import pallas as pl
from jax.experimental.pallas import tpu as pltpu

E = 8
H = 1024
I_DIM = 1408
BM = 128
P_MAX = 5120
NB = P_MAX // BM
EW = 128


def _gate_kernel(x_ref, gw_ref, e0_ref, e1_ref, w0_ref, w1_ref,
                 r0_ref, r1_ref, cnt_ref):
    T = x_ref.shape[0]
    xv = x_ref[...]
    gw = gw_ref[...]
    logits = lax.dot_general(xv, gw, (((1,), (1,)), ((), ())),
                             precision=lax.Precision.HIGHEST)
    eidx = lax.broadcasted_iota(jnp.int32, (T, E), 1)
    m1 = jnp.max(logits, axis=1, keepdims=True)
    e0 = jnp.min(jnp.where(logits == m1, eidx, E), axis=1)
    sel0 = eidx == e0[:, None]
    l2 = jnp.where(sel0, jnp.float32(-1e30), logits)
    m2 = jnp.max(l2, axis=1, keepdims=True)
    e1 = jnp.min(jnp.where(l2 == m2, eidx, E), axis=1)
    eb = jnp.exp(m2[:, 0] - m1[:, 0])
    w0_ref[...] = 1.0 / (1.0 + eb)
    w1_ref[...] = eb / (1.0 + eb)
    e0_ref[...] = e0
    e1_ref[...] = e1
    widx = lax.broadcasted_iota(jnp.int32, (T, EW), 1)
    oh0 = (widx == e0[:, None]).astype(jnp.float32)
    oh1 = (widx == e1[:, None]).astype(jnp.float32)
    ti = lax.broadcasted_iota(jnp.int32, (T, T), 0)
    tj = lax.broadcasted_iota(jnp.int32, (T, T), 1)
    tri = (tj < ti).astype(jnp.float32)
    csum = lax.dot_general(tri, oh0 + oh1, (((1,), (0,)), ((), ())),
                           precision=lax.Precision.HIGHEST)
    r0_ref[...] = jnp.sum(csum * oh0, axis=1).astype(jnp.int32)
    r1_ref[...] = jnp.sum(csum * oh1, axis=1).astype(jnp.int32)
    cnt_ref[...] = jnp.sum(oh0 + oh1, axis=0).astype(jnp.int32)


def _ffn_kernel(be_ref, xg_ref, w1_ref, wu_ref, w2_ref, y_ref):
    del be_ref
    xb = xg_ref[...]
    u = lax.dot_general(xb, w1_ref[0], (((1,), (1,)), ((), ())))
    v = lax.dot_general(xb, wu_ref[0], (((1,), (1,)), ((), ())))
    h = u / (1.0 + jnp.exp(-u)) * v
    y_ref[...] = lax.dot_general(h, w2_ref[0], (((1,), (1,)), ((), ())))


def kernel(x, gate_w, w1, w_up, w2):
    orig_shape = x.shape
    xf = x.reshape(-1, x.shape[-1])
    T = xf.shape[0]

    x16 = xf.astype(jnp.float16).astype(jnp.float32)
    gwf = gate_w.astype(jnp.float32)
    e0, e1, w0, w1g, r0, r1, cnt = pl.pallas_call(
        _gate_kernel,
        out_shape=[
            jax.ShapeDtypeStruct((T,), jnp.int32),
            jax.ShapeDtypeStruct((T,), jnp.int32),
            jax.ShapeDtypeStruct((T,), jnp.float32),
            jax.ShapeDtypeStruct((T,), jnp.float32),
            jax.ShapeDtypeStruct((T,), jnp.int32),
            jax.ShapeDtypeStruct((T,), jnp.int32),
            jax.ShapeDtypeStruct((EW,), jnp.int32),
        ],
    )(x16, gwf)

    cnt8 = cnt[:E]
    padded = ((cnt8 + BM - 1) // BM) * BM
    off = jnp.cumsum(padded) - padded
    pos0 = off[e0] + r0
    pos1 = off[e1] + r1
    tok = jnp.arange(T, dtype=jnp.int32)
    src_idx = jnp.zeros((P_MAX,), jnp.int32).at[pos0].set(tok).at[pos1].set(tok)
    blk = jnp.arange(NB, dtype=jnp.int32) * BM
    block_expert = (jnp.sum(blk[:, None] >= off[None, :], axis=1) - 1).astype(jnp.int32)

    xg = xf[src_idx]

    grid_spec = pltpu.PrefetchScalarGridSpec(
        num_scalar_prefetch=1,
        grid=(NB,),
        in_specs=[
            pl.BlockSpec((BM, H), lambda i, be: (i, 0)),
            pl.BlockSpec((1, I_DIM, H), lambda i, be: (be[i], 0, 0)),
            pl.BlockSpec((1, I_DIM, H), lambda i, be: (be[i], 0, 0)),
            pl.BlockSpec((1, H, I_DIM), lambda i, be: (be[i], 0, 0)),
        ],
        out_specs=pl.BlockSpec((BM, H), lambda i, be: (i, 0)),
    )
    y = pl.pallas_call(
        _ffn_kernel,
        grid_spec=grid_spec,
        out_shape=jax.ShapeDtypeStruct((P_MAX, H), jnp.float32),
    )(block_expert, xg, w1, w_up, w2)

    out = w0[:, None] * y[pos0] + w1g[:, None] * y[pos1]
    return out.reshape(orig_shape)


# speedup vs baseline: 1.0607x; 1.0607x over previous
import functools

import jax
import jax.numpy as jnp
from jax import lax
from jax.experimental import pallas as pl
from jax.experimental.pallas import tpu as pltpu

E = 8
H = 1024
I_DIM = 1408
BM = 128
P_MAX = 5120
NB = P_MAX // BM
EW = 128


def _round_f16(v):
    u = lax.bitcast_convert_type(v, jnp.int32)
    lsb = lax.shift_right_logical(u, 13) & 1
    u = (u + 0xFFF + lsb) & ~0x1FFF
    return lax.bitcast_convert_type(u, jnp.float32)


def _gate_kernel(x_ref, gw_ref, e0_ref, e1_ref, w0_ref, w1_ref,
                 r0_ref, r1_ref, cnt_ref):
    T = x_ref.shape[0]
    xv = x_ref[...]
    gw = gw_ref[...]
    logits = _round_f16(lax.dot_general(xv, gw, (((1,), (1,)), ((), ()))))
    eidx = lax.broadcasted_iota(jnp.int32, (T, E), 1)
    m1 = jnp.max(logits, axis=1, keepdims=True)
    e0 = jnp.min(jnp.where(logits == m1, eidx, E), axis=1)
    sel0 = eidx == e0[:, None]
    l2 = jnp.where(sel0, jnp.float32(-1e30), logits)
    m2 = jnp.max(l2, axis=1, keepdims=True)
    e1 = jnp.min(jnp.where(l2 == m2, eidx, E), axis=1)
    t = _round_f16(m2[:, 0] - m1[:, 0])
    eb = _round_f16(jnp.exp(t))
    z = _round_f16(1.0 + eb)
    w0_ref[...] = _round_f16(1.0 / z)
    w1_ref[...] = _round_f16(eb / z)
    e0_ref[...] = e0
    e1_ref[...] = e1
    widx = lax.broadcasted_iota(jnp.int32, (T, EW), 1)
    oh0 = (widx == e0[:, None]).astype(jnp.float32)
    oh1 = (widx == e1[:, None]).astype(jnp.float32)
    ti = lax.broadcasted_iota(jnp.int32, (T, T), 0)
    tj = lax.broadcasted_iota(jnp.int32, (T, T), 1)
    tri = (tj < ti).astype(jnp.float32)
    csum = lax.dot_general(tri, oh0 + oh1, (((1,), (0,)), ((), ())),
                           precision=lax.Precision.HIGHEST)
    r0_ref[...] = jnp.sum(csum * oh0, axis=1).astype(jnp.int32)
    r1_ref[...] = jnp.sum(csum * oh1, axis=1).astype(jnp.int32)
    cnt_ref[...] = jnp.sum(oh0 + oh1, axis=0).astype(jnp.int32)


def _ffn_kernel(be_ref, xg_ref, w1_ref, wu_ref, w2_ref, y_ref):
    del be_ref
    xb = xg_ref[...]
    u = lax.dot_general(xb, w1_ref[0], (((1,), (1,)), ((), ())))
    v = lax.dot_general(xb, wu_ref[0], (((1,), (1,)), ((), ())))
    h = u / (1.0 + jnp.exp(-u)) * v
    y_ref[...] = lax.dot_general(h, w2_ref[0], (((1,), (1,)), ((), ())))


def kernel(x, gate_w, w1, w_up, w2):
    orig_shape = x.shape
    xf = x.reshape(-1, x.shape[-1])
    T = xf.shape[0]

    x16 = xf.astype(jnp.float16).astype(jnp.float32)
    gwf = gate_w.astype(jnp.float32)
    e0, e1, w0, w1g, r0, r1, cnt = pl.pallas_call(
        _gate_kernel,
        out_shape=[
            jax.ShapeDtypeStruct((T,), jnp.int32),
            jax.ShapeDtypeStruct((T,), jnp.int32),
            jax.ShapeDtypeStruct((T,), jnp.float32),
            jax.ShapeDtypeStruct((T,), jnp.float32),
            jax.ShapeDtypeStruct((T,), jnp.int32),
            jax.ShapeDtypeStruct((T,), jnp.int32),
            jax.ShapeDtypeStruct((EW,), jnp.int32),
        ],
    )(x16, gwf)

    cnt8 = cnt[:E]
    padded = ((cnt8 + BM - 1) // BM) * BM
    off = jnp.cumsum(padded) - padded
    pos0 = off[e0] + r0
    pos1 = off[e1] + r1
    tok = jnp.arange(T, dtype=jnp.int32)
    src_idx = jnp.zeros((P_MAX,), jnp.int32).at[pos0].set(tok).at[pos1].set(tok)
    blk = jnp.arange(NB, dtype=jnp.int32) * BM
    block_expert = (jnp.sum(blk[:, None] >= off[None, :], axis=1) - 1).astype(jnp.int32)

    xg = xf[src_idx]

    grid_spec = pltpu.PrefetchScalarGridSpec(
        num_scalar_prefetch=1,
        grid=(NB,),
        in_specs=[
            pl.BlockSpec((BM, H), lambda i, be: (i, 0)),
            pl.BlockSpec((1, I_DIM, H), lambda i, be: (be[i], 0, 0)),
            pl.BlockSpec((1, I_DIM, H), lambda i, be: (be[i], 0, 0)),
            pl.BlockSpec((1, H, I_DIM), lambda i, be: (be[i], 0, 0)),
        ],
        out_specs=pl.BlockSpec((BM, H), lambda i, be: (i, 0)),
    )
    y = pl.pallas_call(
        _ffn_kernel,
        grid_spec=grid_spec,
        out_shape=jax.ShapeDtypeStruct((P_MAX, H), jnp.float32),
    )(block_expert, xg, w1, w_up, w2)

    out = w0[:, None] * y[pos0] + w1g[:, None] * y[pos1]
    return out.reshape(orig_shape)
